# baseline (speedup 1.0000x reference)
import functools

import jax
import jax.numpy as jnp
from jax import lax
from jax.experimental import pallas as pl
from jax.experimental.pallas import tpu as pltpu
from jax.experimental.pallas import tpu_sc as plsc

LANES = 16


def _build_kernel(n_tokens, hidden, max_pos, chunk, n_workers):
    b_per_w = n_tokens // n_workers
    n_chunks = b_per_w // chunk
    slices_per_row = hidden // LANES
    assert n_chunks % 2 == 0 and n_chunks >= 4

    mesh = plsc.VectorSubcoreMesh(core_axis_name="c", subcore_axis_name="s")

    @functools.partial(
        pl.kernel,
        mesh=mesh,
        out_type=jax.ShapeDtypeStruct((n_tokens, hidden), jnp.float32),
        scratch_types=[
            pltpu.VMEM((b_per_w,), jnp.int32),
            pltpu.VMEM((b_per_w + LANES,), jnp.int32),
            pltpu.VMEM((chunk, hidden), jnp.float32),
            pltpu.VMEM((chunk, hidden), jnp.float32),
            pltpu.VMEM((max_pos * hidden,), jnp.float32),
            pltpu.SemaphoreType.DMA,
            pltpu.SemaphoreType.DMA,
            pltpu.SemaphoreType.DMA,
            pltpu.SemaphoreType.DMA,
        ],
    )
    def k(tok_ids, pos_ids, tok_table, pos_table_flat, out,
          ti_all, pi_all, trows0, trows1, pos_v,
          semt0, semt1, sems0, sems1):
        wid = lax.axis_index("s") * 2 + lax.axis_index("c")
        base = wid * b_per_w
        trows = [trows0, trows1]
        semt = [semt0, semt1]
        sems = [sems0, sems1]

        pltpu.sync_copy(pos_table_flat, pos_v)
        pltpu.sync_copy(tok_ids.at[pl.ds(base, b_per_w)], ti_all)
        pltpu.sync_copy(pos_ids.at[pl.ds(base, b_per_w)],
                        pi_all.at[pl.ds(0, b_per_w)])

        def start_gather(ci, b):
            pltpu.async_copy(
                tok_table.at[ti_all.at[pl.ds(ci * chunk, chunk)]],
                trows[b], semt[b])

        def wait_gather(b):
            pltpu.make_async_copy(
                tok_table.at[ti_all.at[pl.ds(0, chunk)]],
                trows[b], semt[b]).wait()

        def add_rows(ci, b):
            def tok_body(t, c):
                pid = pi_all[pl.ds(ci * chunk + t, LANES)][0]
                off = pid * hidden
                for j in range(slices_per_row):
                    sl = pl.ds(j * LANES, LANES)
                    plsc.addupdate(trows[b].at[t, sl],
                                   pos_v[pl.ds(off + j * LANES, LANES)])
                return c
            lax.fori_loop(0, chunk, tok_body, None)

        def start_store(ci, b):
            pltpu.async_copy(
                trows[b], out.at[pl.ds(base + ci * chunk, chunk)], sems[b])

        def wait_store(b):
            pltpu.make_async_copy(
                trows[b], out.at[pl.ds(base, chunk)], sems[b]).wait()

        start_gather(0, 0)
        start_gather(1, 1)
        wait_gather(0)
        add_rows(0, 0)
        start_store(0, 0)

        def pair_body(p, carry):
            for b in (1, 0):
                ci = 2 * p + 2 - b
                wait_store(1 - b)
                start_gather(ci + 1, 1 - b)
                wait_gather(b)
                add_rows(ci, b)
                start_store(ci, b)
            return carry

        lax.fori_loop(0, (n_chunks - 2) // 2, pair_body, None)

        ci = n_chunks - 1
        wait_store(0)
        wait_gather(1)
        add_rows(ci, 1)
        start_store(ci, 1)
        wait_store(1)

    return k


def kernel(input_ids, position_ids, token_embedding, position_embedding):
    batch, seq = input_ids.shape
    vocab, hidden = token_embedding.shape
    max_pos = position_embedding.shape[0]
    n_tokens = batch * seq

    n_workers = 32
    chunk = 32
    assert n_tokens % (n_workers * 2 * chunk) == 0

    tok_flat = input_ids.T.reshape(n_tokens).astype(jnp.int32)
    pos_flat = position_ids.T.reshape(n_tokens).astype(jnp.int32)
    k = _build_kernel(n_tokens, hidden, max_pos, chunk, n_workers)
    out = k(tok_flat, pos_flat, token_embedding,
            position_embedding.reshape(max_pos * hidden))
    return out.reshape(seq, batch, hidden).transpose(1, 0, 2)


# speedup vs baseline: 1.1023x; 1.1023x over previous
import functools

import jax
import jax.numpy as jnp
from jax import lax
from jax.experimental import pallas as pl
from jax.experimental.pallas import tpu as pltpu
from jax.experimental.pallas import tpu_sc as plsc

LANES = 16


def _build_kernel(n_tokens, hidden, chunk, n_workers):
    b_per_w = n_tokens // n_workers
    n_chunks = b_per_w // chunk
    slices_per_row = hidden // LANES
    nsplit = 4
    sub = chunk // nsplit
    assert n_chunks % 2 == 0 and n_chunks >= 4 and sub % 8 == 0

    mesh = plsc.VectorSubcoreMesh(core_axis_name="c", subcore_axis_name="s")

    @functools.partial(
        pl.kernel,
        mesh=mesh,
        out_type=jax.ShapeDtypeStruct((n_tokens, hidden), jnp.float32),
        scratch_types=[
            pltpu.VMEM((b_per_w,), jnp.int32),
            pltpu.VMEM((b_per_w,), jnp.int32),
            pltpu.VMEM((chunk, hidden), jnp.float32),
            pltpu.VMEM((chunk, hidden), jnp.float32),
            pltpu.VMEM((chunk, hidden), jnp.float32),
            pltpu.VMEM((chunk, hidden), jnp.float32),
            pltpu.SemaphoreType.DMA,
            pltpu.SemaphoreType.DMA,
            pltpu.SemaphoreType.DMA,
            pltpu.SemaphoreType.DMA,
            pltpu.SemaphoreType.DMA,
            pltpu.SemaphoreType.DMA,
        ],
    )
    def k(tok_ids, pos_ids, tok_table, pos_table, out,
          ti_all, pi_all, trows0, trows1, prows0, prows1,
          semt0, semt1, semp0, semp1, sems0, sems1):
        wid = lax.axis_index("s") * 2 + lax.axis_index("c")
        base = wid * b_per_w
        trows = [trows0, trows1]
        prows = [prows0, prows1]
        semt = [semt0, semt1]
        semp = [semp0, semp1]
        sems = [sems0, sems1]

        pltpu.sync_copy(tok_ids.at[pl.ds(base, b_per_w)], ti_all)
        pltpu.sync_copy(pos_ids.at[pl.ds(base, b_per_w)], pi_all)

        def start_gathers(ci, b):
            for si in range(nsplit):
                pltpu.async_copy(
                    tok_table.at[ti_all.at[pl.ds(ci * chunk + si * sub, sub)]],
                    trows[b].at[pl.ds(si * sub, sub)], semt[b])
                pltpu.async_copy(
                    pos_table.at[pi_all.at[pl.ds(ci * chunk + si * sub, sub)]],
                    prows[b].at[pl.ds(si * sub, sub)], semp[b])

        def wait_gathers(b):
            for si in range(nsplit):
                pltpu.make_async_copy(
                    tok_table.at[ti_all.at[pl.ds(0, sub)]],
                    trows[b].at[pl.ds(si * sub, sub)], semt[b]).wait()
                pltpu.make_async_copy(
                    pos_table.at[pi_all.at[pl.ds(0, sub)]],
                    prows[b].at[pl.ds(si * sub, sub)], semp[b]).wait()

        def add_rows(b):
            def tok_body(t, c):
                for j in range(slices_per_row):
                    sl = pl.ds(j * LANES, LANES)
                    plsc.addupdate(trows[b].at[t, sl], prows[b][t, sl])
                return c
            lax.fori_loop(0, chunk, tok_body, None)

        def start_store(ci, b):
            pltpu.async_copy(
                trows[b], out.at[pl.ds(base + ci * chunk, chunk)], sems[b])

        def wait_store(b):
            pltpu.make_async_copy(
                trows[b], out.at[pl.ds(base, chunk)], sems[b]).wait()

        start_gathers(0, 0)
        start_gathers(1, 1)
        wait_gathers(0)
        add_rows(0)
        start_store(0, 0)

        def pair_body(p, carry):
            for b in (1, 0):
                ci = 2 * p + 2 - b
                wait_store(1 - b)
                start_gathers(ci + 1, 1 - b)
                wait_gathers(b)
                add_rows(b)
                start_store(ci, b)
            return carry

        lax.fori_loop(0, (n_chunks - 2) // 2, pair_body, None)

        ci = n_chunks - 1
        wait_store(0)
        wait_gathers(1)
        add_rows(1)
        start_store(ci, 1)
        wait_store(1)

    return k


def kernel(input_ids, position_ids, token_embedding, position_embedding):
    batch, seq = input_ids.shape
    vocab, hidden = token_embedding.shape
    n_tokens = batch * seq

    n_workers = 32
    chunk = 32
    assert n_tokens % (n_workers * 2 * chunk) == 0

    tok_flat = input_ids.T.reshape(n_tokens).astype(jnp.int32)
    pos_flat = position_ids.T.reshape(n_tokens).astype(jnp.int32)
    k = _build_kernel(n_tokens, hidden, chunk, n_workers)
    out = k(tok_flat, pos_flat, token_embedding, position_embedding)
    return out.reshape(seq, batch, hidden).transpose(1, 0, 2)


# speedup vs baseline: 2.0128x; 1.8261x over previous
import functools

import jax
import jax.numpy as jnp
from jax import lax
from jax.experimental import pallas as pl
from jax.experimental.pallas import tpu as pltpu
from jax.experimental.pallas import tpu_sc as plsc

LANES = 16


def _build_kernel(n_tokens, hidden, chunk, n_workers):
    b_per_w = n_tokens // n_workers
    n_chunks = b_per_w // chunk
    slices_per_row = hidden // LANES
    nsplit = 4
    sub = chunk // nsplit
    assert n_chunks % 2 == 0 and n_chunks >= 4 and sub % 8 == 0

    mesh = plsc.VectorSubcoreMesh(core_axis_name="c", subcore_axis_name="s")

    @functools.partial(
        pl.kernel,
        mesh=mesh,
        out_type=jax.ShapeDtypeStruct((n_tokens, hidden), jnp.float32),
        scratch_types=[
            pltpu.VMEM((b_per_w,), jnp.int32),
            pltpu.VMEM((b_per_w,), jnp.int32),
            pltpu.VMEM((chunk, hidden), jnp.float32),
            pltpu.VMEM((chunk, hidden), jnp.float32),
            pltpu.VMEM((chunk, hidden), jnp.float32),
            pltpu.VMEM((chunk, hidden), jnp.float32),
            pltpu.SemaphoreType.DMA,
            pltpu.SemaphoreType.DMA,
            pltpu.SemaphoreType.DMA,
            pltpu.SemaphoreType.DMA,
            pltpu.SemaphoreType.DMA,
            pltpu.SemaphoreType.DMA,
        ],
    )
    def k(tok_ids, pos_ids, tok_table, pos_table, out,
          ti_all, pi_all, trows0, trows1, prows0, prows1,
          semt0, semt1, semp0, semp1, sems0, sems1):
        wid = lax.axis_index("s") * 2 + lax.axis_index("c")
        base = wid * b_per_w
        trows = [trows0, trows1]
        prows = [prows0, prows1]
        semt = [semt0, semt1]
        semp = [semp0, semp1]
        sems = [sems0, sems1]

        pltpu.sync_copy(tok_ids.at[pl.ds(base, b_per_w)], ti_all)
        pltpu.sync_copy(pos_ids.at[pl.ds(base, b_per_w)], pi_all)

        def start_gathers(ci, b):
            for si in range(nsplit):
                pltpu.async_copy(
                    tok_table.at[ti_all.at[pl.ds(ci * chunk + si * sub, sub)]],
                    trows[b].at[pl.ds(si * sub, sub)], semt[b])
                pltpu.async_copy(
                    pos_table.at[pi_all.at[pl.ds(ci * chunk + si * sub, sub)]],
                    prows[b].at[pl.ds(si * sub, sub)], semp[b])

        def wait_gathers(b):
            for si in range(nsplit):
                pltpu.make_async_copy(
                    tok_table.at[ti_all.at[pl.ds(0, sub)]],
                    trows[b].at[pl.ds(si * sub, sub)], semt[b]).wait()
                pltpu.make_async_copy(
                    pos_table.at[pi_all.at[pl.ds(0, sub)]],
                    prows[b].at[pl.ds(si * sub, sub)], semp[b]).wait()

        def add_rows(b):
            def tok_body(t, c):
                for j in range(slices_per_row):
                    sl = pl.ds(j * LANES, LANES)
                    plsc.addupdate(trows[b].at[t, sl], prows[b][t, sl])
                return c
            lax.fori_loop(0, chunk, tok_body, None)

        def start_store(ci, b):
            pltpu.async_copy(
                trows[b], out.at[pl.ds(base + ci * chunk, chunk)], sems[b])

        def wait_store(b):
            pltpu.make_async_copy(
                trows[b], out.at[pl.ds(base, chunk)], sems[b]).wait()

        start_gathers(0, 0)
        start_gathers(1, 1)
        wait_gathers(0)
        add_rows(0)
        start_store(0, 0)

        def pair_body(p, carry):
            for b in (1, 0):
                ci = 2 * p + 2 - b
                wait_store(1 - b)
                start_gathers(ci + 1, 1 - b)
                wait_gathers(b)
                add_rows(b)
                start_store(ci, b)
            return carry

        lax.fori_loop(0, (n_chunks - 2) // 2, pair_body, None)

        ci = n_chunks - 1
        wait_store(0)
        wait_gathers(1)
        add_rows(1)
        start_store(ci, 1)
        wait_store(1)

    return k


def kernel(input_ids, position_ids, token_embedding, position_embedding):
    batch, seq = input_ids.shape
    vocab, hidden = token_embedding.shape
    n_tokens = batch * seq

    n_workers = 32
    chunk = 32
    assert n_tokens % (n_workers * 2 * chunk) == 0

    tok_flat = input_ids.T.reshape(n_tokens).astype(jnp.int32)
    pos_flat = position_ids.T.reshape(n_tokens).astype(jnp.int32)
    n_rep = 32
    max_pos = position_embedding.shape[0]
    pos_rep = jnp.tile(position_embedding, (n_rep, 1))
    pos_flat = pos_flat + (jnp.arange(n_tokens, dtype=jnp.int32) % n_rep) * max_pos
    k = _build_kernel(n_tokens, hidden, chunk, n_workers)
    out = k(tok_flat, pos_flat, token_embedding, pos_rep)
    return out.reshape(seq, batch, hidden).transpose(1, 0, 2)
